# baseline (speedup 1.0000x reference)
import functools

import jax
import jax.numpy as jnp
from jax import lax
from jax.experimental import pallas as pl
from jax.experimental.pallas import tpu as pltpu
from jax.experimental.pallas import tpu_sc as plsc

N_NODES = 10000
FEATS = 128
HFEATS = FEATS // 2
N_EDGES = 320000

NC, NS, L = 2, 16, 16
NW = NC * NS
CHUNK = 125
A_EPW = N_EDGES // NS
A_NCHUNK = A_EPW // CHUNK
D_EPW = N_EDGES // NW
D_NCHUNK = D_EPW // CHUNK
NPAD = 10240
RPS = NPAD // NS
ZROWS = 128


def _agg_kernel_body(f0, f1, srcs, dsts, acc_out,
                     src_v, dst_v, rows0, rows1, zbuf, acc_sh,
                     gsem0, gsem1, ssem0, ssem1):
    c = lax.axis_index("c")
    s = lax.axis_index("s")
    base = s * RPS

    pltpu.sync_copy(srcs.at[s], src_v)
    pltpu.sync_copy(dsts.at[s], dst_v)

    zero16 = jnp.zeros((L,), jnp.float32)

    def zloop(i, carry):
        for j in range(HFEATS // L):
            zbuf[i, pl.ds(j * L, L)] = zero16
        return carry

    lax.fori_loop(0, ZROWS, zloop, 0)
    for k in range(RPS // ZROWS):
        pltpu.sync_copy(zbuf, acc_sh.at[pl.ds(base + k * ZROWS, ZROWS)])

    plsc.subcore_barrier()

    rows = (rows0, rows1)
    gsem = (gsem0, gsem1)
    ssem = (ssem0, ssem1)

    def start_gather(j, b):
        @pl.when(c == 0)
        def _():
            pltpu.async_copy(f0.at[src_v.at[j]], rows[b], gsem[b])

        @pl.when(c == 1)
        def _():
            pltpu.async_copy(f1.at[src_v.at[j]], rows[b], gsem[b])

    def wait_gather(b):
        pltpu.make_async_copy(f0.at[src_v.at[0]], rows[b], gsem[b]).wait()

    def start_scatter(j, b):
        pltpu.async_copy(rows[b], acc_sh.at[dst_v.at[j]], ssem[b], add=True)

    def wait_scatter(b):
        pltpu.make_async_copy(rows[b], acc_sh.at[dst_v.at[0]],
                              ssem[b]).wait()

    start_gather(0, 0)
    start_gather(1, 1)

    def chunk_pair(i, carry):
        j = 2 * i
        for b in range(2):
            wait_gather(b)
            start_scatter(j + b, b)

        @pl.when(i + 1 < A_NCHUNK // 2)
        def _():
            for b in range(2):
                wait_scatter(b)
                start_gather(j + 2 + b, b)

        return carry

    lax.fori_loop(0, A_NCHUNK // 2, chunk_pair, 0)
    for b in range(2):
        wait_scatter(b)

    plsc.subcore_barrier()

    pltpu.sync_copy(acc_sh.at[pl.ds(base, RPS)],
                    acc_out.at[c, pl.ds(base, RPS)])


def _agg_deg_kernel_body(f0, f1, srcs, dsts, acc_out, deg_out,
                         src_v, dst_v, rows0, rows1, ones_v, zbuf, dzbuf,
                         acc_sh, deg_sh, sem0, sem1):
    c = lax.axis_index("c")
    s = lax.axis_index("s")
    base = s * RPS

    pltpu.sync_copy(srcs.at[s], src_v)
    pltpu.sync_copy(dsts.at[s], dst_v)

    zero16 = jnp.zeros((L,), jnp.float32)

    def zloop(i, carry):
        for j in range(HFEATS // L):
            zbuf[i, pl.ds(j * L, L)] = zero16
        return carry

    lax.fori_loop(0, ZROWS, zloop, 0)
    for k in range(RPS // ZROWS):
        pltpu.sync_copy(zbuf, acc_sh.at[pl.ds(base + k * ZROWS, ZROWS)])

    def dzloop(i, carry):
        dzbuf[i, :] = zero16
        return carry

    lax.fori_loop(0, ZROWS, dzloop, 0)
    for k in range(RPS // ZROWS):
        pltpu.sync_copy(dzbuf, deg_sh.at[pl.ds(base + k * ZROWS, ZROWS)])

    one16 = jnp.ones((L,), jnp.float32)

    def oloop(i, carry):
        ones_v[i, :] = one16
        return carry

    lax.fori_loop(0, CHUNK, oloop, 0)

    plsc.subcore_barrier()

    def start_gather(j, rows, sem):
        @pl.when(c == 0)
        def _():
            pltpu.async_copy(f0.at[src_v.at[j]], rows, sem)

        @pl.when(c == 1)
        def _():
            pltpu.async_copy(f1.at[src_v.at[j]], rows, sem)

    def wait_gather(rows, sem):
        pltpu.make_async_copy(f0.at[src_v.at[0]], rows, sem).wait()

    start_gather(0, rows0, sem0)

    def chunk_pair(i, carry):
        j = 2 * i
        start_gather(j + 1, rows1, sem1)

        @pl.when(c == 0)
        def _():
            pltpu.sync_copy(ones_v, deg_sh.at[dst_v.at[j]], add=True)

        @pl.when(c == 1)
        def _():
            pltpu.sync_copy(ones_v, deg_sh.at[dst_v.at[j + 1]], add=True)

        wait_gather(rows0, sem0)
        pltpu.sync_copy(rows0, acc_sh.at[dst_v.at[j]], add=True)

        @pl.when(i + 1 < A_NCHUNK // 2)
        def _():
            start_gather(j + 2, rows0, sem0)

        wait_gather(rows1, sem1)
        pltpu.sync_copy(rows1, acc_sh.at[dst_v.at[j + 1]], add=True)
        return carry

    lax.fori_loop(0, A_NCHUNK // 2, chunk_pair, 0)

    plsc.subcore_barrier()

    pltpu.sync_copy(acc_sh.at[pl.ds(base, RPS)],
                    acc_out.at[c, pl.ds(base, RPS)])
    pltpu.sync_copy(deg_sh.at[pl.ds(base, RPS)],
                    deg_out.at[c, pl.ds(base, RPS)])


def _deg_kernel_body(dsts, deg_out, dst_v, ones_v, dbuf, deg_sh):
    c = lax.axis_index("c")
    s = lax.axis_index("s")
    w = c * NS + s
    base = s * RPS

    pltpu.sync_copy(dsts.at[w], dst_v)

    zero16 = jnp.zeros((L,), jnp.float32)

    def dzloop(i, carry):
        dbuf[i, :] = zero16
        return carry

    lax.fori_loop(0, RPS, dzloop, 0)
    pltpu.sync_copy(dbuf, deg_sh.at[pl.ds(base, RPS)])

    one16 = jnp.ones((L,), jnp.float32)

    def oloop(i, carry):
        ones_v[i, :] = one16
        return carry

    lax.fori_loop(0, CHUNK, oloop, 0)

    plsc.subcore_barrier()

    def chunk_body(j, carry):
        pltpu.sync_copy(ones_v, deg_sh.at[dst_v.at[j]], add=True)
        return carry

    lax.fori_loop(0, D_NCHUNK, chunk_body, 0)

    plsc.subcore_barrier()

    pltpu.sync_copy(deg_sh.at[pl.ds(base, RPS)],
                    deg_out.at[c, pl.ds(base, RPS)])


@functools.lru_cache(maxsize=None)
def _make_agg():
    mesh = plsc.VectorSubcoreMesh(core_axis_name="c", subcore_axis_name="s",
                                  num_cores=NC, num_subcores=NS)
    return pl.kernel(
        _agg_kernel_body,
        out_type=jax.ShapeDtypeStruct((NC, NPAD, HFEATS), jnp.float32),
        mesh=mesh,
        compiler_params=pltpu.CompilerParams(use_tc_tiling_on_sc=False),
        scratch_types=[
            pltpu.VMEM((A_NCHUNK, CHUNK), jnp.int32),
            pltpu.VMEM((A_NCHUNK, CHUNK), jnp.int32),
            pltpu.VMEM((CHUNK, HFEATS), jnp.float32),
            pltpu.VMEM((CHUNK, HFEATS), jnp.float32),
            pltpu.VMEM((ZROWS, HFEATS), jnp.float32),
            pltpu.VMEM_SHARED((NPAD, HFEATS), jnp.float32),
            pltpu.SemaphoreType.DMA,
            pltpu.SemaphoreType.DMA,
            pltpu.SemaphoreType.DMA,
            pltpu.SemaphoreType.DMA,
        ],
    )


@functools.lru_cache(maxsize=None)
def _make_agg_deg():
    mesh = plsc.VectorSubcoreMesh(core_axis_name="c", subcore_axis_name="s",
                                  num_cores=NC, num_subcores=NS)
    return pl.kernel(
        _agg_deg_kernel_body,
        out_type=[jax.ShapeDtypeStruct((NC, NPAD, HFEATS), jnp.float32),
                  jax.ShapeDtypeStruct((NC, NPAD, L), jnp.float32)],
        mesh=mesh,
        compiler_params=pltpu.CompilerParams(use_tc_tiling_on_sc=False),
        scratch_types=[
            pltpu.VMEM((A_NCHUNK, CHUNK), jnp.int32),
            pltpu.VMEM((A_NCHUNK, CHUNK), jnp.int32),
            pltpu.VMEM((CHUNK, HFEATS), jnp.float32),
            pltpu.VMEM((CHUNK, HFEATS), jnp.float32),
            pltpu.VMEM((CHUNK, L), jnp.float32),
            pltpu.VMEM((ZROWS, HFEATS), jnp.float32),
            pltpu.VMEM((ZROWS, L), jnp.float32),
            pltpu.VMEM_SHARED((NPAD, HFEATS), jnp.float32),
            pltpu.VMEM_SHARED((NPAD, L), jnp.float32),
            pltpu.SemaphoreType.DMA,
            pltpu.SemaphoreType.DMA,
        ],
    )


@functools.lru_cache(maxsize=None)
def _make_deg():
    mesh = plsc.VectorSubcoreMesh(core_axis_name="c", subcore_axis_name="s",
                                  num_cores=NC, num_subcores=NS)
    return pl.kernel(
        _deg_kernel_body,
        out_type=jax.ShapeDtypeStruct((NC, NPAD, L), jnp.float32),
        mesh=mesh,
        compiler_params=pltpu.CompilerParams(use_tc_tiling_on_sc=False),
        scratch_types=[
            pltpu.VMEM((D_NCHUNK, CHUNK), jnp.int32),
            pltpu.VMEM((CHUNK, L), jnp.float32),
            pltpu.VMEM((RPS, L), jnp.float32),
            pltpu.VMEM_SHARED((NPAD, L), jnp.float32),
        ],
    )


def _dense_body(relu, p_ref, d_ref, x_ref, wl_ref, bl_ref, wr_ref, o_ref):
    p = jnp.concatenate([p_ref[0], p_ref[1]], axis=1)
    d = d_ref[0] + d_ref[1]
    deg = jnp.maximum(d[:, 0:1], 1.0)
    mean = p / deg
    xf = jnp.concatenate([x_ref[0], x_ref[1]], axis=1)
    for t in range(2):
        acc = lax.dot_general(mean, wl_ref[t], (((1,), (1,)), ((), ())),
                              preferred_element_type=jnp.float32)
        acc = acc + bl_ref[t]
        acc = acc + lax.dot_general(xf, wr_ref[t], (((1,), (1,)), ((), ())),
                                    preferred_element_type=jnp.float32)
        if relu:
            acc = jnp.maximum(acc, 0.0)
        o_ref[t] = acc


def _dense(parts, deg, xs, Wl, bl, Wr, relu):
    R = 1000
    grid = (N_NODES // R,)
    return pl.pallas_call(
        functools.partial(_dense_body, relu),
        grid=grid,
        in_specs=[
            pl.BlockSpec((2, R, HFEATS), lambda i: (0, i, 0)),
            pl.BlockSpec((2, R, L), lambda i: (0, i, 0)),
            pl.BlockSpec((2, R, HFEATS), lambda i: (0, i, 0)),
            pl.BlockSpec((2, HFEATS, FEATS), lambda i: (0, 0, 0)),
            pl.BlockSpec((2, 1, HFEATS), lambda i: (0, 0, 0)),
            pl.BlockSpec((2, HFEATS, FEATS), lambda i: (0, 0, 0)),
        ],
        out_specs=pl.BlockSpec((2, R, HFEATS), lambda i: (0, i, 0)),
        out_shape=jax.ShapeDtypeStruct((2, N_NODES, HFEATS), jnp.float32),
    )(parts, deg, xs, Wl, bl, Wr)


def _split_w(W):
    return W.reshape(2, HFEATS, FEATS)


def _split_b(b):
    return b.reshape(2, 1, HFEATS)


def kernel(x, edge_index, W1l, b1l, W1r, W2l, b2l, W2r):
    xs = jnp.stack([x[:, :HFEATS], x[:, HFEATS:]])
    srcs_a = edge_index[0].reshape(NS, A_NCHUNK, CHUNK)
    dsts_a = edge_index[1].reshape(NS, A_NCHUNK, CHUNK)
    p1, deg = _make_agg_deg()(xs[0], xs[1], srcs_a, dsts_a)
    hs = _dense(p1, deg, xs, _split_w(W1l), _split_b(b1l), _split_w(W1r),
                relu=True)
    p2 = _make_agg()(hs[0], hs[1], srcs_a, dsts_a)
    os = _dense(p2, deg, hs, _split_w(W2l), _split_b(b2l), _split_w(W2r),
                relu=False)
    return jnp.concatenate([os[0], os[1]], axis=1)


# speedup vs baseline: 1.0896x; 1.0896x over previous
import functools

import jax
import jax.numpy as jnp
from jax import lax
from jax.experimental import pallas as pl
from jax.experimental.pallas import tpu as pltpu
from jax.experimental.pallas import tpu_sc as plsc

N_NODES = 10000
FEATS = 128
HFEATS = FEATS // 2
N_EDGES = 320000

NC, NS, L = 2, 16, 16
NW = NC * NS
CHUNK = 125
A_EPW = N_EDGES // NS
A_NCHUNK = A_EPW // CHUNK
D_EPW = N_EDGES // NW
D_NCHUNK = D_EPW // CHUNK
NPAD = 10240
RPS = NPAD // NS
ZROWS = 128


def _agg_kernel_body(f0, f1, srcs, dsts, acc_out,
                     src_v, dst_v, rows0, rows1, zbuf, acc_sh, sem0, sem1):
    c = lax.axis_index("c")
    s = lax.axis_index("s")
    base = s * RPS

    pltpu.sync_copy(srcs.at[s], src_v)
    pltpu.sync_copy(dsts.at[s], dst_v)

    zero16 = jnp.zeros((L,), jnp.float32)

    def zloop(i, carry):
        for j in range(HFEATS // L):
            zbuf[i, pl.ds(j * L, L)] = zero16
        return carry

    lax.fori_loop(0, ZROWS, zloop, 0)
    for k in range(RPS // ZROWS):
        pltpu.sync_copy(zbuf, acc_sh.at[pl.ds(base + k * ZROWS, ZROWS)])

    plsc.subcore_barrier()

    def start_gather(j, rows, sem):
        @pl.when(c == 0)
        def _():
            pltpu.async_copy(f0.at[src_v.at[j]], rows, sem)

        @pl.when(c == 1)
        def _():
            pltpu.async_copy(f1.at[src_v.at[j]], rows, sem)

    def wait_gather(rows, sem):
        pltpu.make_async_copy(f0.at[src_v.at[0]], rows, sem).wait()

    start_gather(0, rows0, sem0)

    def chunk_pair(i, carry):
        j = 2 * i
        start_gather(j + 1, rows1, sem1)
        wait_gather(rows0, sem0)
        pltpu.sync_copy(rows0, acc_sh.at[dst_v.at[j]], add=True)

        @pl.when(i + 1 < A_NCHUNK // 2)
        def _():
            start_gather(j + 2, rows0, sem0)

        wait_gather(rows1, sem1)
        pltpu.sync_copy(rows1, acc_sh.at[dst_v.at[j + 1]], add=True)
        return carry

    lax.fori_loop(0, A_NCHUNK // 2, chunk_pair, 0)

    plsc.subcore_barrier()

    pltpu.sync_copy(acc_sh.at[pl.ds(base, RPS)],
                    acc_out.at[c, pl.ds(base, RPS)])


def _agg_deg_kernel_body(f0, f1, srcs, dsts, acc_out, deg_out,
                         src_v, dst_v, rows0, rows1, ones_v, zbuf, dzbuf,
                         acc_sh, deg_sh, sem0, sem1):
    c = lax.axis_index("c")
    s = lax.axis_index("s")
    base = s * RPS

    pltpu.sync_copy(srcs.at[s], src_v)
    pltpu.sync_copy(dsts.at[s], dst_v)

    zero16 = jnp.zeros((L,), jnp.float32)

    def zloop(i, carry):
        for j in range(HFEATS // L):
            zbuf[i, pl.ds(j * L, L)] = zero16
        return carry

    lax.fori_loop(0, ZROWS, zloop, 0)
    for k in range(RPS // ZROWS):
        pltpu.sync_copy(zbuf, acc_sh.at[pl.ds(base + k * ZROWS, ZROWS)])

    def dzloop(i, carry):
        dzbuf[i, :] = zero16
        return carry

    lax.fori_loop(0, ZROWS, dzloop, 0)
    for k in range(RPS // ZROWS):
        pltpu.sync_copy(dzbuf, deg_sh.at[pl.ds(base + k * ZROWS, ZROWS)])

    one16 = jnp.ones((L,), jnp.float32)

    def oloop(i, carry):
        ones_v[i, :] = one16
        return carry

    lax.fori_loop(0, CHUNK, oloop, 0)

    plsc.subcore_barrier()

    def start_gather(j, rows, sem):
        @pl.when(c == 0)
        def _():
            pltpu.async_copy(f0.at[src_v.at[j]], rows, sem)

        @pl.when(c == 1)
        def _():
            pltpu.async_copy(f1.at[src_v.at[j]], rows, sem)

    def wait_gather(rows, sem):
        pltpu.make_async_copy(f0.at[src_v.at[0]], rows, sem).wait()

    start_gather(0, rows0, sem0)

    def chunk_pair(i, carry):
        j = 2 * i
        start_gather(j + 1, rows1, sem1)

        @pl.when(c == 0)
        def _():
            pltpu.sync_copy(ones_v, deg_sh.at[dst_v.at[j]], add=True)

        @pl.when(c == 1)
        def _():
            pltpu.sync_copy(ones_v, deg_sh.at[dst_v.at[j + 1]], add=True)

        wait_gather(rows0, sem0)
        pltpu.sync_copy(rows0, acc_sh.at[dst_v.at[j]], add=True)

        @pl.when(i + 1 < A_NCHUNK // 2)
        def _():
            start_gather(j + 2, rows0, sem0)

        wait_gather(rows1, sem1)
        pltpu.sync_copy(rows1, acc_sh.at[dst_v.at[j + 1]], add=True)
        return carry

    lax.fori_loop(0, A_NCHUNK // 2, chunk_pair, 0)

    plsc.subcore_barrier()

    pltpu.sync_copy(acc_sh.at[pl.ds(base, RPS)],
                    acc_out.at[c, pl.ds(base, RPS)])
    pltpu.sync_copy(deg_sh.at[pl.ds(base, RPS)],
                    deg_out.at[c, pl.ds(base, RPS)])


def _deg_kernel_body(dsts, deg_out, dst_v, ones_v, dbuf, deg_sh):
    c = lax.axis_index("c")
    s = lax.axis_index("s")
    w = c * NS + s
    base = s * RPS

    pltpu.sync_copy(dsts.at[w], dst_v)

    zero16 = jnp.zeros((L,), jnp.float32)

    def dzloop(i, carry):
        dbuf[i, :] = zero16
        return carry

    lax.fori_loop(0, RPS, dzloop, 0)
    pltpu.sync_copy(dbuf, deg_sh.at[pl.ds(base, RPS)])

    one16 = jnp.ones((L,), jnp.float32)

    def oloop(i, carry):
        ones_v[i, :] = one16
        return carry

    lax.fori_loop(0, CHUNK, oloop, 0)

    plsc.subcore_barrier()

    def chunk_body(j, carry):
        pltpu.sync_copy(ones_v, deg_sh.at[dst_v.at[j]], add=True)
        return carry

    lax.fori_loop(0, D_NCHUNK, chunk_body, 0)

    plsc.subcore_barrier()

    pltpu.sync_copy(deg_sh.at[pl.ds(base, RPS)],
                    deg_out.at[c, pl.ds(base, RPS)])


@functools.lru_cache(maxsize=None)
def _make_agg():
    mesh = plsc.VectorSubcoreMesh(core_axis_name="c", subcore_axis_name="s",
                                  num_cores=NC, num_subcores=NS)
    return pl.kernel(
        _agg_kernel_body,
        out_type=jax.ShapeDtypeStruct((NC, NPAD, HFEATS), jnp.float32),
        mesh=mesh,
        compiler_params=pltpu.CompilerParams(use_tc_tiling_on_sc=False),
        scratch_types=[
            pltpu.VMEM((A_NCHUNK, CHUNK), jnp.int32),
            pltpu.VMEM((A_NCHUNK, CHUNK), jnp.int32),
            pltpu.VMEM((CHUNK, HFEATS), jnp.float32),
            pltpu.VMEM((CHUNK, HFEATS), jnp.float32),
            pltpu.VMEM((ZROWS, HFEATS), jnp.float32),
            pltpu.VMEM_SHARED((NPAD, HFEATS), jnp.float32),
            pltpu.SemaphoreType.DMA,
            pltpu.SemaphoreType.DMA,
        ],
    )


@functools.lru_cache(maxsize=None)
def _make_agg_deg():
    mesh = plsc.VectorSubcoreMesh(core_axis_name="c", subcore_axis_name="s",
                                  num_cores=NC, num_subcores=NS)
    return pl.kernel(
        _agg_deg_kernel_body,
        out_type=[jax.ShapeDtypeStruct((NC, NPAD, HFEATS), jnp.float32),
                  jax.ShapeDtypeStruct((NC, NPAD, L), jnp.float32)],
        mesh=mesh,
        compiler_params=pltpu.CompilerParams(use_tc_tiling_on_sc=False),
        scratch_types=[
            pltpu.VMEM((A_NCHUNK, CHUNK), jnp.int32),
            pltpu.VMEM((A_NCHUNK, CHUNK), jnp.int32),
            pltpu.VMEM((CHUNK, HFEATS), jnp.float32),
            pltpu.VMEM((CHUNK, HFEATS), jnp.float32),
            pltpu.VMEM((CHUNK, L), jnp.float32),
            pltpu.VMEM((ZROWS, HFEATS), jnp.float32),
            pltpu.VMEM((ZROWS, L), jnp.float32),
            pltpu.VMEM_SHARED((NPAD, HFEATS), jnp.float32),
            pltpu.VMEM_SHARED((NPAD, L), jnp.float32),
            pltpu.SemaphoreType.DMA,
            pltpu.SemaphoreType.DMA,
        ],
    )


@functools.lru_cache(maxsize=None)
def _make_deg():
    mesh = plsc.VectorSubcoreMesh(core_axis_name="c", subcore_axis_name="s",
                                  num_cores=NC, num_subcores=NS)
    return pl.kernel(
        _deg_kernel_body,
        out_type=jax.ShapeDtypeStruct((NC, NPAD, L), jnp.float32),
        mesh=mesh,
        compiler_params=pltpu.CompilerParams(use_tc_tiling_on_sc=False),
        scratch_types=[
            pltpu.VMEM((D_NCHUNK, CHUNK), jnp.int32),
            pltpu.VMEM((CHUNK, L), jnp.float32),
            pltpu.VMEM((RPS, L), jnp.float32),
            pltpu.VMEM_SHARED((NPAD, L), jnp.float32),
        ],
    )


def _dense_body(relu, p_ref, d_ref, x_ref, wl_ref, bl_ref, wr_ref, o_ref):
    p = jnp.concatenate([p_ref[0], p_ref[1]], axis=1)
    d = d_ref[0] + d_ref[1]
    deg = jnp.maximum(d[:, 0:1], 1.0)
    mean = p / deg
    xf = jnp.concatenate([x_ref[0], x_ref[1]], axis=1)
    for t in range(2):
        acc = lax.dot_general(mean, wl_ref[t], (((1,), (1,)), ((), ())),
                              preferred_element_type=jnp.float32)
        acc = acc + bl_ref[t]
        acc = acc + lax.dot_general(xf, wr_ref[t], (((1,), (1,)), ((), ())),
                                    preferred_element_type=jnp.float32)
        if relu:
            acc = jnp.maximum(acc, 0.0)
        o_ref[t] = acc


def _dense(parts, deg, xs, Wl, bl, Wr, relu):
    R = 1000
    grid = (N_NODES // R,)
    return pl.pallas_call(
        functools.partial(_dense_body, relu),
        grid=grid,
        in_specs=[
            pl.BlockSpec((2, R, HFEATS), lambda i: (0, i, 0)),
            pl.BlockSpec((2, R, L), lambda i: (0, i, 0)),
            pl.BlockSpec((2, R, HFEATS), lambda i: (0, i, 0)),
            pl.BlockSpec((2, HFEATS, FEATS), lambda i: (0, 0, 0)),
            pl.BlockSpec((2, 1, HFEATS), lambda i: (0, 0, 0)),
            pl.BlockSpec((2, HFEATS, FEATS), lambda i: (0, 0, 0)),
        ],
        out_specs=pl.BlockSpec((2, R, HFEATS), lambda i: (0, i, 0)),
        out_shape=jax.ShapeDtypeStruct((2, N_NODES, HFEATS), jnp.float32),
    )(parts, deg, xs, Wl, bl, Wr)


def _split_w(W):
    return W.reshape(2, HFEATS, FEATS)


def _split_b(b):
    return b.reshape(2, 1, HFEATS)


def kernel(x, edge_index, W1l, b1l, W1r, W2l, b2l, W2r):
    xs = jnp.stack([x[:, :HFEATS], x[:, HFEATS:]])
    srcs_a = edge_index[0].reshape(NS, A_NCHUNK, CHUNK)
    dsts_a = edge_index[1].reshape(NS, A_NCHUNK, CHUNK)
    p1, deg = _make_agg_deg()(xs[0], xs[1], srcs_a, dsts_a)
    hs = _dense(p1, deg, xs, _split_w(W1l), _split_b(b1l), _split_w(W1r),
                relu=True)
    p2 = _make_agg()(hs[0], hs[1], srcs_a, dsts_a)
    os = _dense(p2, deg, hs, _split_w(W2l), _split_b(b2l), _split_w(W2r),
                relu=False)
    return jnp.concatenate([os[0], os[1]], axis=1)


# speedup vs baseline: 1.1111x; 1.0197x over previous
import functools

import jax
import jax.numpy as jnp
from jax import lax
from jax.experimental import pallas as pl
from jax.experimental.pallas import tpu as pltpu
from jax.experimental.pallas import tpu_sc as plsc

N_NODES = 10000
FEATS = 128
HFEATS = FEATS // 2
N_EDGES = 320000

NC, NS, L = 2, 16, 16
NW = NC * NS
CHUNK = 125
A_EPW = N_EDGES // NS
A_NCHUNK = A_EPW // CHUNK
D_EPW = N_EDGES // NW
D_NCHUNK = D_EPW // CHUNK
NPAD = 10240
RPS = NPAD // NS
ZROWS = 128


def _agg_kernel_body(f0, f1, srcs, dsts, acc_out,
                     src_v, dst_v, rows0, rows1, zbuf, acc_sh, sem0, sem1):
    c = lax.axis_index("c")
    s = lax.axis_index("s")
    base = s * RPS

    pltpu.async_copy(srcs.at[s], src_v, sem0)
    pltpu.async_copy(dsts.at[s], dst_v, sem1)

    zero16 = jnp.zeros((L,), jnp.float32)

    def zloop(i, carry):
        for j in range(HFEATS // L):
            zbuf[i, pl.ds(j * L, L)] = zero16
        return carry

    lax.fori_loop(0, ZROWS, zloop, 0)
    for k in range(RPS // ZROWS):
        pltpu.sync_copy(zbuf, acc_sh.at[pl.ds(base + k * ZROWS, ZROWS)])

    pltpu.make_async_copy(srcs.at[s], src_v, sem0).wait()
    pltpu.make_async_copy(dsts.at[s], dst_v, sem1).wait()

    plsc.subcore_barrier()

    def start_gather(j, rows, sem):
        @pl.when(c == 0)
        def _():
            pltpu.async_copy(f0.at[src_v.at[j]], rows, sem)

        @pl.when(c == 1)
        def _():
            pltpu.async_copy(f1.at[src_v.at[j]], rows, sem)

    def wait_gather(rows, sem):
        pltpu.make_async_copy(f0.at[src_v.at[0]], rows, sem).wait()

    start_gather(0, rows0, sem0)

    def chunk_pair(i, carry):
        j = 2 * i
        start_gather(j + 1, rows1, sem1)
        wait_gather(rows0, sem0)
        pltpu.sync_copy(rows0, acc_sh.at[dst_v.at[j]], add=True)

        @pl.when(i + 1 < A_NCHUNK // 2)
        def _():
            start_gather(j + 2, rows0, sem0)

        wait_gather(rows1, sem1)
        pltpu.sync_copy(rows1, acc_sh.at[dst_v.at[j + 1]], add=True)
        return carry

    lax.fori_loop(0, A_NCHUNK // 2, chunk_pair, 0)

    plsc.subcore_barrier()

    pltpu.sync_copy(acc_sh.at[pl.ds(base, RPS)],
                    acc_out.at[c, pl.ds(base, RPS)])


def _agg_deg_kernel_body(f0, f1, srcs, dsts, acc_out, deg_out,
                         src_v, dst_v, rows0, rows1, ones_v, zbuf, dzbuf,
                         acc_sh, deg_sh, sem0, sem1):
    c = lax.axis_index("c")
    s = lax.axis_index("s")
    base = s * RPS

    pltpu.async_copy(srcs.at[s], src_v, sem0)
    pltpu.async_copy(dsts.at[s], dst_v, sem1)

    zero16 = jnp.zeros((L,), jnp.float32)

    def zloop(i, carry):
        for j in range(HFEATS // L):
            zbuf[i, pl.ds(j * L, L)] = zero16
        return carry

    lax.fori_loop(0, ZROWS, zloop, 0)
    for k in range(RPS // ZROWS):
        pltpu.sync_copy(zbuf, acc_sh.at[pl.ds(base + k * ZROWS, ZROWS)])

    def dzloop(i, carry):
        dzbuf[i, :] = zero16
        return carry

    lax.fori_loop(0, ZROWS, dzloop, 0)
    for k in range(RPS // ZROWS):
        pltpu.sync_copy(dzbuf, deg_sh.at[pl.ds(base + k * ZROWS, ZROWS)])

    one16 = jnp.ones((L,), jnp.float32)

    def oloop(i, carry):
        ones_v[i, :] = one16
        return carry

    lax.fori_loop(0, CHUNK, oloop, 0)

    pltpu.make_async_copy(srcs.at[s], src_v, sem0).wait()
    pltpu.make_async_copy(dsts.at[s], dst_v, sem1).wait()

    plsc.subcore_barrier()

    def start_gather(j, rows, sem):
        @pl.when(c == 0)
        def _():
            pltpu.async_copy(f0.at[src_v.at[j]], rows, sem)

        @pl.when(c == 1)
        def _():
            pltpu.async_copy(f1.at[src_v.at[j]], rows, sem)

    def wait_gather(rows, sem):
        pltpu.make_async_copy(f0.at[src_v.at[0]], rows, sem).wait()

    start_gather(0, rows0, sem0)

    def chunk_pair(i, carry):
        j = 2 * i
        start_gather(j + 1, rows1, sem1)

        @pl.when(c == 0)
        def _():
            pltpu.sync_copy(ones_v, deg_sh.at[dst_v.at[j]], add=True)

        @pl.when(c == 1)
        def _():
            pltpu.sync_copy(ones_v, deg_sh.at[dst_v.at[j + 1]], add=True)

        wait_gather(rows0, sem0)
        pltpu.sync_copy(rows0, acc_sh.at[dst_v.at[j]], add=True)

        @pl.when(i + 1 < A_NCHUNK // 2)
        def _():
            start_gather(j + 2, rows0, sem0)

        wait_gather(rows1, sem1)
        pltpu.sync_copy(rows1, acc_sh.at[dst_v.at[j + 1]], add=True)
        return carry

    lax.fori_loop(0, A_NCHUNK // 2, chunk_pair, 0)

    plsc.subcore_barrier()

    pltpu.sync_copy(acc_sh.at[pl.ds(base, RPS)],
                    acc_out.at[c, pl.ds(base, RPS)])
    pltpu.sync_copy(deg_sh.at[pl.ds(base, RPS)],
                    deg_out.at[c, pl.ds(base, RPS)])


def _deg_kernel_body(dsts, deg_out, dst_v, ones_v, dbuf, deg_sh):
    c = lax.axis_index("c")
    s = lax.axis_index("s")
    w = c * NS + s
    base = s * RPS

    pltpu.sync_copy(dsts.at[w], dst_v)

    zero16 = jnp.zeros((L,), jnp.float32)

    def dzloop(i, carry):
        dbuf[i, :] = zero16
        return carry

    lax.fori_loop(0, RPS, dzloop, 0)
    pltpu.sync_copy(dbuf, deg_sh.at[pl.ds(base, RPS)])

    one16 = jnp.ones((L,), jnp.float32)

    def oloop(i, carry):
        ones_v[i, :] = one16
        return carry

    lax.fori_loop(0, CHUNK, oloop, 0)

    plsc.subcore_barrier()

    def chunk_body(j, carry):
        pltpu.sync_copy(ones_v, deg_sh.at[dst_v.at[j]], add=True)
        return carry

    lax.fori_loop(0, D_NCHUNK, chunk_body, 0)

    plsc.subcore_barrier()

    pltpu.sync_copy(deg_sh.at[pl.ds(base, RPS)],
                    deg_out.at[c, pl.ds(base, RPS)])


@functools.lru_cache(maxsize=None)
def _make_agg():
    mesh = plsc.VectorSubcoreMesh(core_axis_name="c", subcore_axis_name="s",
                                  num_cores=NC, num_subcores=NS)
    return pl.kernel(
        _agg_kernel_body,
        out_type=jax.ShapeDtypeStruct((NC, NPAD, HFEATS), jnp.float32),
        mesh=mesh,
        compiler_params=pltpu.CompilerParams(use_tc_tiling_on_sc=False),
        scratch_types=[
            pltpu.VMEM((A_NCHUNK, CHUNK), jnp.int32),
            pltpu.VMEM((A_NCHUNK, CHUNK), jnp.int32),
            pltpu.VMEM((CHUNK, HFEATS), jnp.float32),
            pltpu.VMEM((CHUNK, HFEATS), jnp.float32),
            pltpu.VMEM((ZROWS, HFEATS), jnp.float32),
            pltpu.VMEM_SHARED((NPAD, HFEATS), jnp.float32),
            pltpu.SemaphoreType.DMA,
            pltpu.SemaphoreType.DMA,
        ],
    )


@functools.lru_cache(maxsize=None)
def _make_agg_deg():
    mesh = plsc.VectorSubcoreMesh(core_axis_name="c", subcore_axis_name="s",
                                  num_cores=NC, num_subcores=NS)
    return pl.kernel(
        _agg_deg_kernel_body,
        out_type=[jax.ShapeDtypeStruct((NC, NPAD, HFEATS), jnp.float32),
                  jax.ShapeDtypeStruct((NC, NPAD, L), jnp.float32)],
        mesh=mesh,
        compiler_params=pltpu.CompilerParams(use_tc_tiling_on_sc=False),
        scratch_types=[
            pltpu.VMEM((A_NCHUNK, CHUNK), jnp.int32),
            pltpu.VMEM((A_NCHUNK, CHUNK), jnp.int32),
            pltpu.VMEM((CHUNK, HFEATS), jnp.float32),
            pltpu.VMEM((CHUNK, HFEATS), jnp.float32),
            pltpu.VMEM((CHUNK, L), jnp.float32),
            pltpu.VMEM((ZROWS, HFEATS), jnp.float32),
            pltpu.VMEM((ZROWS, L), jnp.float32),
            pltpu.VMEM_SHARED((NPAD, HFEATS), jnp.float32),
            pltpu.VMEM_SHARED((NPAD, L), jnp.float32),
            pltpu.SemaphoreType.DMA,
            pltpu.SemaphoreType.DMA,
        ],
    )


@functools.lru_cache(maxsize=None)
def _make_deg():
    mesh = plsc.VectorSubcoreMesh(core_axis_name="c", subcore_axis_name="s",
                                  num_cores=NC, num_subcores=NS)
    return pl.kernel(
        _deg_kernel_body,
        out_type=jax.ShapeDtypeStruct((NC, NPAD, L), jnp.float32),
        mesh=mesh,
        compiler_params=pltpu.CompilerParams(use_tc_tiling_on_sc=False),
        scratch_types=[
            pltpu.VMEM((D_NCHUNK, CHUNK), jnp.int32),
            pltpu.VMEM((CHUNK, L), jnp.float32),
            pltpu.VMEM((RPS, L), jnp.float32),
            pltpu.VMEM_SHARED((NPAD, L), jnp.float32),
        ],
    )


def _dense_body(relu, p_ref, d_ref, x_ref, wl_ref, bl_ref, wr_ref, o_ref):
    p = jnp.concatenate([p_ref[0], p_ref[1]], axis=1)
    d = d_ref[0] + d_ref[1]
    deg = jnp.maximum(d[:, 0:1], 1.0)
    mean = p / deg
    xf = jnp.concatenate([x_ref[0], x_ref[1]], axis=1)
    for t in range(2):
        acc = lax.dot_general(mean, wl_ref[t], (((1,), (1,)), ((), ())),
                              preferred_element_type=jnp.float32)
        acc = acc + bl_ref[t]
        acc = acc + lax.dot_general(xf, wr_ref[t], (((1,), (1,)), ((), ())),
                                    preferred_element_type=jnp.float32)
        if relu:
            acc = jnp.maximum(acc, 0.0)
        o_ref[t] = acc


def _dense(parts, deg, xs, Wl, bl, Wr, relu):
    R = 1000
    grid = (N_NODES // R,)
    return pl.pallas_call(
        functools.partial(_dense_body, relu),
        grid=grid,
        in_specs=[
            pl.BlockSpec((2, R, HFEATS), lambda i: (0, i, 0)),
            pl.BlockSpec((2, R, L), lambda i: (0, i, 0)),
            pl.BlockSpec((2, R, HFEATS), lambda i: (0, i, 0)),
            pl.BlockSpec((2, HFEATS, FEATS), lambda i: (0, 0, 0)),
            pl.BlockSpec((2, 1, HFEATS), lambda i: (0, 0, 0)),
            pl.BlockSpec((2, HFEATS, FEATS), lambda i: (0, 0, 0)),
        ],
        out_specs=pl.BlockSpec((2, R, HFEATS), lambda i: (0, i, 0)),
        out_shape=jax.ShapeDtypeStruct((2, N_NODES, HFEATS), jnp.float32),
    )(parts, deg, xs, Wl, bl, Wr)


def _split_w(W):
    return W.reshape(2, HFEATS, FEATS)


def _split_b(b):
    return b.reshape(2, 1, HFEATS)


def kernel(x, edge_index, W1l, b1l, W1r, W2l, b2l, W2r):
    xs = jnp.stack([x[:, :HFEATS], x[:, HFEATS:]])
    srcs_a = edge_index[0].reshape(NS, A_NCHUNK, CHUNK)
    dsts_a = edge_index[1].reshape(NS, A_NCHUNK, CHUNK)
    p1, deg = _make_agg_deg()(xs[0], xs[1], srcs_a, dsts_a)
    hs = _dense(p1, deg, xs, _split_w(W1l), _split_b(b1l), _split_w(W1r),
                relu=True)
    p2 = _make_agg()(hs[0], hs[1], srcs_a, dsts_a)
    os = _dense(p2, deg, hs, _split_w(W2l), _split_b(b2l), _split_w(W2r),
                relu=False)
    return jnp.concatenate([os[0], os[1]], axis=1)


# speedup vs baseline: 1.1230x; 1.0107x over previous
import functools

import jax
import jax.numpy as jnp
from jax import lax
from jax.experimental import pallas as pl
from jax.experimental.pallas import tpu as pltpu
from jax.experimental.pallas import tpu_sc as plsc

N_NODES = 10000
FEATS = 128
HFEATS = FEATS // 2
N_EDGES = 320000

NC, NS, L = 2, 16, 16
NW = NC * NS
CHUNK = 125
A_EPW = N_EDGES // NS
A_NCHUNK = A_EPW // CHUNK
D_EPW = N_EDGES // NW
D_NCHUNK = D_EPW // CHUNK
NPAD = 10240
RPS = NPAD // NS
ZROWS = 128


def _agg_kernel_body(f0, f1, srcs, dsts, acc_out,
                     src_v, dst_v, rows0, rows1, zbuf, acc_sh, sem0, sem1):
    c = lax.axis_index("c")
    s = lax.axis_index("s")
    base = s * RPS

    pltpu.async_copy(srcs.at[s], src_v, sem0)
    pltpu.async_copy(dsts.at[s], dst_v, sem1)

    zero16 = jnp.zeros((L,), jnp.float32)

    def zloop(i, carry):
        for j in range(HFEATS // L):
            zbuf[i, pl.ds(j * L, L)] = zero16
        return carry

    lax.fori_loop(0, ZROWS, zloop, 0)
    for k in range(RPS // ZROWS):
        pltpu.sync_copy(zbuf, acc_sh.at[pl.ds(base + k * ZROWS, ZROWS)])

    pltpu.make_async_copy(srcs.at[s], src_v, sem0).wait()
    pltpu.make_async_copy(dsts.at[s], dst_v, sem1).wait()

    plsc.subcore_barrier()

    def start_gather(j, rows, sem):
        @pl.when(c == 0)
        def _():
            pltpu.async_copy(f0.at[src_v.at[j]], rows, sem)

        @pl.when(c == 1)
        def _():
            pltpu.async_copy(f1.at[src_v.at[j]], rows, sem)

    def wait_gather(rows, sem):
        pltpu.make_async_copy(f0.at[src_v.at[0]], rows, sem).wait()

    start_gather(0, rows0, sem0)

    def chunk_pair(i, carry):
        j = 2 * i
        start_gather(j + 1, rows1, sem1)
        wait_gather(rows0, sem0)
        pltpu.sync_copy(rows0, acc_sh.at[dst_v.at[j]], add=True)

        @pl.when(i + 1 < A_NCHUNK // 2)
        def _():
            start_gather(j + 2, rows0, sem0)

        wait_gather(rows1, sem1)
        pltpu.sync_copy(rows1, acc_sh.at[dst_v.at[j + 1]], add=True)
        return carry

    lax.fori_loop(0, A_NCHUNK // 2, chunk_pair, 0)

    plsc.subcore_barrier()

    pltpu.sync_copy(acc_sh.at[pl.ds(base, RPS)],
                    acc_out.at[c, pl.ds(base, RPS)])


def _agg_deg_kernel_body(f0, f1, srcs, dsts, acc_out, deg_out,
                         src_v, dst_v, rows0, rows1, ones_v, zbuf, dzbuf,
                         acc_sh, deg_sh, sem0, sem1):
    c = lax.axis_index("c")
    s = lax.axis_index("s")
    base = s * RPS

    pltpu.async_copy(srcs.at[s], src_v, sem0)
    pltpu.async_copy(dsts.at[s], dst_v, sem1)

    zero16 = jnp.zeros((L,), jnp.float32)

    def zloop(i, carry):
        for j in range(HFEATS // L):
            zbuf[i, pl.ds(j * L, L)] = zero16
        return carry

    lax.fori_loop(0, ZROWS, zloop, 0)
    for k in range(RPS // ZROWS):
        pltpu.sync_copy(zbuf, acc_sh.at[pl.ds(base + k * ZROWS, ZROWS)])

    def dzloop(i, carry):
        dzbuf[i, :] = zero16
        return carry

    lax.fori_loop(0, ZROWS, dzloop, 0)
    for k in range(RPS // ZROWS):
        pltpu.sync_copy(dzbuf, deg_sh.at[pl.ds(base + k * ZROWS, ZROWS)])

    one16 = jnp.ones((L,), jnp.float32)

    def oloop(i, carry):
        ones_v[i, :] = one16
        return carry

    lax.fori_loop(0, CHUNK, oloop, 0)

    pltpu.make_async_copy(srcs.at[s], src_v, sem0).wait()
    pltpu.make_async_copy(dsts.at[s], dst_v, sem1).wait()

    plsc.subcore_barrier()

    def start_gather(j, rows, sem):
        @pl.when(c == 0)
        def _():
            pltpu.async_copy(f0.at[src_v.at[j]], rows, sem)

        @pl.when(c == 1)
        def _():
            pltpu.async_copy(f1.at[src_v.at[j]], rows, sem)

    def wait_gather(rows, sem):
        pltpu.make_async_copy(f0.at[src_v.at[0]], rows, sem).wait()

    start_gather(0, rows0, sem0)

    def chunk_pair(i, carry):
        j = 2 * i
        start_gather(j + 1, rows1, sem1)

        @pl.when(c == 0)
        def _():
            pltpu.sync_copy(ones_v, deg_sh.at[dst_v.at[j]], add=True)

        @pl.when(c == 1)
        def _():
            pltpu.sync_copy(ones_v, deg_sh.at[dst_v.at[j + 1]], add=True)

        wait_gather(rows0, sem0)
        pltpu.sync_copy(rows0, acc_sh.at[dst_v.at[j]], add=True)

        @pl.when(i + 1 < A_NCHUNK // 2)
        def _():
            start_gather(j + 2, rows0, sem0)

        wait_gather(rows1, sem1)
        pltpu.sync_copy(rows1, acc_sh.at[dst_v.at[j + 1]], add=True)
        return carry

    lax.fori_loop(0, A_NCHUNK // 2, chunk_pair, 0)

    plsc.subcore_barrier()

    pltpu.sync_copy(acc_sh.at[pl.ds(base, RPS)],
                    acc_out.at[c, pl.ds(base, RPS)])
    pltpu.sync_copy(deg_sh.at[pl.ds(base, RPS)],
                    deg_out.at[c, pl.ds(base, RPS)])


def _deg_kernel_body(dsts, deg_out, dst_v, ones_v, dbuf, deg_sh):
    c = lax.axis_index("c")
    s = lax.axis_index("s")
    w = c * NS + s
    base = s * RPS

    pltpu.sync_copy(dsts.at[w], dst_v)

    zero16 = jnp.zeros((L,), jnp.float32)

    def dzloop(i, carry):
        dbuf[i, :] = zero16
        return carry

    lax.fori_loop(0, RPS, dzloop, 0)
    pltpu.sync_copy(dbuf, deg_sh.at[pl.ds(base, RPS)])

    one16 = jnp.ones((L,), jnp.float32)

    def oloop(i, carry):
        ones_v[i, :] = one16
        return carry

    lax.fori_loop(0, CHUNK, oloop, 0)

    plsc.subcore_barrier()

    def chunk_body(j, carry):
        pltpu.sync_copy(ones_v, deg_sh.at[dst_v.at[j]], add=True)
        return carry

    lax.fori_loop(0, D_NCHUNK, chunk_body, 0)

    plsc.subcore_barrier()

    pltpu.sync_copy(deg_sh.at[pl.ds(base, RPS)],
                    deg_out.at[c, pl.ds(base, RPS)])


@functools.lru_cache(maxsize=None)
def _make_agg():
    mesh = plsc.VectorSubcoreMesh(core_axis_name="c", subcore_axis_name="s",
                                  num_cores=NC, num_subcores=NS)
    return pl.kernel(
        _agg_kernel_body,
        out_type=jax.ShapeDtypeStruct((NC, NPAD, HFEATS), jnp.float32),
        mesh=mesh,
        compiler_params=pltpu.CompilerParams(use_tc_tiling_on_sc=False),
        scratch_types=[
            pltpu.VMEM((A_NCHUNK, CHUNK), jnp.int32),
            pltpu.VMEM((A_NCHUNK, CHUNK), jnp.int32),
            pltpu.VMEM((CHUNK, HFEATS), jnp.float32),
            pltpu.VMEM((CHUNK, HFEATS), jnp.float32),
            pltpu.VMEM((ZROWS, HFEATS), jnp.float32),
            pltpu.VMEM_SHARED((NPAD, HFEATS), jnp.float32),
            pltpu.SemaphoreType.DMA,
            pltpu.SemaphoreType.DMA,
        ],
    )


@functools.lru_cache(maxsize=None)
def _make_agg_deg():
    mesh = plsc.VectorSubcoreMesh(core_axis_name="c", subcore_axis_name="s",
                                  num_cores=NC, num_subcores=NS)
    return pl.kernel(
        _agg_deg_kernel_body,
        out_type=[jax.ShapeDtypeStruct((NC, NPAD, HFEATS), jnp.float32),
                  jax.ShapeDtypeStruct((NC, NPAD, L), jnp.float32)],
        mesh=mesh,
        compiler_params=pltpu.CompilerParams(use_tc_tiling_on_sc=False),
        scratch_types=[
            pltpu.VMEM((A_NCHUNK, CHUNK), jnp.int32),
            pltpu.VMEM((A_NCHUNK, CHUNK), jnp.int32),
            pltpu.VMEM((CHUNK, HFEATS), jnp.float32),
            pltpu.VMEM((CHUNK, HFEATS), jnp.float32),
            pltpu.VMEM((CHUNK, L), jnp.float32),
            pltpu.VMEM((ZROWS, HFEATS), jnp.float32),
            pltpu.VMEM((ZROWS, L), jnp.float32),
            pltpu.VMEM_SHARED((NPAD, HFEATS), jnp.float32),
            pltpu.VMEM_SHARED((NPAD, L), jnp.float32),
            pltpu.SemaphoreType.DMA,
            pltpu.SemaphoreType.DMA,
        ],
    )


@functools.lru_cache(maxsize=None)
def _make_deg():
    mesh = plsc.VectorSubcoreMesh(core_axis_name="c", subcore_axis_name="s",
                                  num_cores=NC, num_subcores=NS)
    return pl.kernel(
        _deg_kernel_body,
        out_type=jax.ShapeDtypeStruct((NC, NPAD, L), jnp.float32),
        mesh=mesh,
        compiler_params=pltpu.CompilerParams(use_tc_tiling_on_sc=False),
        scratch_types=[
            pltpu.VMEM((D_NCHUNK, CHUNK), jnp.int32),
            pltpu.VMEM((CHUNK, L), jnp.float32),
            pltpu.VMEM((RPS, L), jnp.float32),
            pltpu.VMEM_SHARED((NPAD, L), jnp.float32),
        ],
    )


def _dense_body(relu, p_ref, d_ref, x_ref, wl_ref, bl_ref, wr_ref, o_ref):
    p = jnp.concatenate([p_ref[0], p_ref[1]], axis=1)
    d = d_ref[0] + d_ref[1]
    deg = jnp.maximum(d[:, 0:1], 1.0)
    mean = p / deg
    xf = jnp.concatenate([x_ref[0], x_ref[1]], axis=1)
    for t in range(2):
        acc = lax.dot_general(mean, wl_ref[t], (((1,), (1,)), ((), ())),
                              preferred_element_type=jnp.float32)
        acc = acc + bl_ref[t]
        acc = acc + lax.dot_general(xf, wr_ref[t], (((1,), (1,)), ((), ())),
                                    preferred_element_type=jnp.float32)
        if relu:
            acc = jnp.maximum(acc, 0.0)
        o_ref[t] = acc


def _dense(parts, deg, xs, Wl, bl, Wr, relu):
    R = 2000
    grid = (N_NODES // R,)
    return pl.pallas_call(
        functools.partial(_dense_body, relu),
        grid=grid,
        in_specs=[
            pl.BlockSpec((2, R, HFEATS), lambda i: (0, i, 0)),
            pl.BlockSpec((2, R, L), lambda i: (0, i, 0)),
            pl.BlockSpec((2, R, HFEATS), lambda i: (0, i, 0)),
            pl.BlockSpec((2, HFEATS, FEATS), lambda i: (0, 0, 0)),
            pl.BlockSpec((2, 1, HFEATS), lambda i: (0, 0, 0)),
            pl.BlockSpec((2, HFEATS, FEATS), lambda i: (0, 0, 0)),
        ],
        out_specs=pl.BlockSpec((2, R, HFEATS), lambda i: (0, i, 0)),
        out_shape=jax.ShapeDtypeStruct((2, N_NODES, HFEATS), jnp.float32),
    )(parts, deg, xs, Wl, bl, Wr)


def _split_w(W):
    return W.reshape(2, HFEATS, FEATS)


def _split_b(b):
    return b.reshape(2, 1, HFEATS)


def kernel(x, edge_index, W1l, b1l, W1r, W2l, b2l, W2r):
    xs = jnp.stack([x[:, :HFEATS], x[:, HFEATS:]])
    srcs_a = edge_index[0].reshape(NS, A_NCHUNK, CHUNK)
    dsts_a = edge_index[1].reshape(NS, A_NCHUNK, CHUNK)
    p1, deg = _make_agg_deg()(xs[0], xs[1], srcs_a, dsts_a)
    hs = _dense(p1, deg, xs, _split_w(W1l), _split_b(b1l), _split_w(W1r),
                relu=True)
    p2 = _make_agg()(hs[0], hs[1], srcs_a, dsts_a)
    os = _dense(p2, deg, hs, _split_w(W2l), _split_b(b2l), _split_w(W2r),
                relu=False)
    return jnp.concatenate([os[0], os[1]], axis=1)


# speedup vs baseline: 1.1231x; 1.0001x over previous
import functools

import jax
import jax.numpy as jnp
from jax import lax
from jax.experimental import pallas as pl
from jax.experimental.pallas import tpu as pltpu
from jax.experimental.pallas import tpu_sc as plsc

N_NODES = 10000
FEATS = 128
HFEATS = FEATS // 2
N_EDGES = 320000

NC, NS, L = 2, 16, 16
CHUNK = 125
A_EPW = N_EDGES // NS
A_NCHUNK = A_EPW // CHUNK
NPAD = 10240
RPS = NPAD // NS
ZROWS = 128


def _agg_kernel_body(f0, f1, srcs, dsts, acc_out,
                     src_v, dst_v, rows0, rows1, zbuf, acc_sh, sem0, sem1):
    c = lax.axis_index("c")
    s = lax.axis_index("s")
    base = s * RPS

    pltpu.async_copy(srcs.at[s], src_v, sem0)
    pltpu.async_copy(dsts.at[s], dst_v, sem1)

    zero16 = jnp.zeros((L,), jnp.float32)

    def zloop(i, carry):
        for j in range(HFEATS // L):
            zbuf[i, pl.ds(j * L, L)] = zero16
        return carry

    lax.fori_loop(0, ZROWS, zloop, 0)
    for k in range(RPS // ZROWS):
        pltpu.sync_copy(zbuf, acc_sh.at[pl.ds(base + k * ZROWS, ZROWS)])

    pltpu.make_async_copy(srcs.at[s], src_v, sem0).wait()
    pltpu.make_async_copy(dsts.at[s], dst_v, sem1).wait()

    plsc.subcore_barrier()

    def start_gather(j, rows, sem):
        @pl.when(c == 0)
        def _():
            pltpu.async_copy(f0.at[src_v.at[j]], rows, sem)

        @pl.when(c == 1)
        def _():
            pltpu.async_copy(f1.at[src_v.at[j]], rows, sem)

    def wait_gather(rows, sem):
        pltpu.make_async_copy(f0.at[src_v.at[0]], rows, sem).wait()

    start_gather(0, rows0, sem0)

    def chunk_pair(i, carry):
        j = 2 * i
        start_gather(j + 1, rows1, sem1)
        wait_gather(rows0, sem0)
        pltpu.sync_copy(rows0, acc_sh.at[dst_v.at[j]], add=True)

        @pl.when(i + 1 < A_NCHUNK // 2)
        def _():
            start_gather(j + 2, rows0, sem0)

        wait_gather(rows1, sem1)
        pltpu.sync_copy(rows1, acc_sh.at[dst_v.at[j + 1]], add=True)
        return carry

    lax.fori_loop(0, A_NCHUNK // 2, chunk_pair, 0)

    plsc.subcore_barrier()

    pltpu.sync_copy(acc_sh.at[pl.ds(base, RPS)],
                    acc_out.at[c, pl.ds(base, RPS)])


def _agg_deg_kernel_body(f0, f1, srcs, dsts, acc_out, deg_out,
                         src_v, dst_v, rows0, rows1, ones_v, zbuf, dzbuf,
                         acc_sh, deg_sh, sem0, sem1):
    c = lax.axis_index("c")
    s = lax.axis_index("s")
    base = s * RPS

    pltpu.async_copy(srcs.at[s], src_v, sem0)
    pltpu.async_copy(dsts.at[s], dst_v, sem1)

    zero16 = jnp.zeros((L,), jnp.float32)

    def zloop(i, carry):
        for j in range(HFEATS // L):
            zbuf[i, pl.ds(j * L, L)] = zero16
        return carry

    lax.fori_loop(0, ZROWS, zloop, 0)
    for k in range(RPS // ZROWS):
        pltpu.sync_copy(zbuf, acc_sh.at[pl.ds(base + k * ZROWS, ZROWS)])

    def dzloop(i, carry):
        dzbuf[i, :] = zero16
        return carry

    lax.fori_loop(0, ZROWS, dzloop, 0)
    for k in range(RPS // ZROWS):
        pltpu.sync_copy(dzbuf, deg_sh.at[pl.ds(base + k * ZROWS, ZROWS)])

    one16 = jnp.ones((L,), jnp.float32)

    def oloop(i, carry):
        ones_v[i, :] = one16
        return carry

    lax.fori_loop(0, CHUNK, oloop, 0)

    pltpu.make_async_copy(srcs.at[s], src_v, sem0).wait()
    pltpu.make_async_copy(dsts.at[s], dst_v, sem1).wait()

    plsc.subcore_barrier()

    def start_gather(j, rows, sem):
        @pl.when(c == 0)
        def _():
            pltpu.async_copy(f0.at[src_v.at[j]], rows, sem)

        @pl.when(c == 1)
        def _():
            pltpu.async_copy(f1.at[src_v.at[j]], rows, sem)

    def wait_gather(rows, sem):
        pltpu.make_async_copy(f0.at[src_v.at[0]], rows, sem).wait()

    start_gather(0, rows0, sem0)

    def chunk_pair(i, carry):
        j = 2 * i
        start_gather(j + 1, rows1, sem1)

        @pl.when(c == 0)
        def _():
            pltpu.sync_copy(ones_v, deg_sh.at[dst_v.at[j]], add=True)

        @pl.when(c == 1)
        def _():
            pltpu.sync_copy(ones_v, deg_sh.at[dst_v.at[j + 1]], add=True)

        wait_gather(rows0, sem0)
        pltpu.sync_copy(rows0, acc_sh.at[dst_v.at[j]], add=True)

        @pl.when(i + 1 < A_NCHUNK // 2)
        def _():
            start_gather(j + 2, rows0, sem0)

        wait_gather(rows1, sem1)
        pltpu.sync_copy(rows1, acc_sh.at[dst_v.at[j + 1]], add=True)
        return carry

    lax.fori_loop(0, A_NCHUNK // 2, chunk_pair, 0)

    plsc.subcore_barrier()

    pltpu.sync_copy(acc_sh.at[pl.ds(base, RPS)],
                    acc_out.at[c, pl.ds(base, RPS)])
    pltpu.sync_copy(deg_sh.at[pl.ds(base, RPS)],
                    deg_out.at[c, pl.ds(base, RPS)])


@functools.lru_cache(maxsize=None)
def _make_agg():
    mesh = plsc.VectorSubcoreMesh(core_axis_name="c", subcore_axis_name="s",
                                  num_cores=NC, num_subcores=NS)
    return pl.kernel(
        _agg_kernel_body,
        out_type=jax.ShapeDtypeStruct((NC, NPAD, HFEATS), jnp.float32),
        mesh=mesh,
        compiler_params=pltpu.CompilerParams(use_tc_tiling_on_sc=False),
        scratch_types=[
            pltpu.VMEM((A_NCHUNK, CHUNK), jnp.int32),
            pltpu.VMEM((A_NCHUNK, CHUNK), jnp.int32),
            pltpu.VMEM((CHUNK, HFEATS), jnp.float32),
            pltpu.VMEM((CHUNK, HFEATS), jnp.float32),
            pltpu.VMEM((ZROWS, HFEATS), jnp.float32),
            pltpu.VMEM_SHARED((NPAD, HFEATS), jnp.float32),
            pltpu.SemaphoreType.DMA,
            pltpu.SemaphoreType.DMA,
        ],
    )


@functools.lru_cache(maxsize=None)
def _make_agg_deg():
    mesh = plsc.VectorSubcoreMesh(core_axis_name="c", subcore_axis_name="s",
                                  num_cores=NC, num_subcores=NS)
    return pl.kernel(
        _agg_deg_kernel_body,
        out_type=[jax.ShapeDtypeStruct((NC, NPAD, HFEATS), jnp.float32),
                  jax.ShapeDtypeStruct((NC, NPAD, L), jnp.float32)],
        mesh=mesh,
        compiler_params=pltpu.CompilerParams(use_tc_tiling_on_sc=False),
        scratch_types=[
            pltpu.VMEM((A_NCHUNK, CHUNK), jnp.int32),
            pltpu.VMEM((A_NCHUNK, CHUNK), jnp.int32),
            pltpu.VMEM((CHUNK, HFEATS), jnp.float32),
            pltpu.VMEM((CHUNK, HFEATS), jnp.float32),
            pltpu.VMEM((CHUNK, L), jnp.float32),
            pltpu.VMEM((ZROWS, HFEATS), jnp.float32),
            pltpu.VMEM((ZROWS, L), jnp.float32),
            pltpu.VMEM_SHARED((NPAD, HFEATS), jnp.float32),
            pltpu.VMEM_SHARED((NPAD, L), jnp.float32),
            pltpu.SemaphoreType.DMA,
            pltpu.SemaphoreType.DMA,
        ],
    )


def _dense_body(relu, p_ref, d_ref, x_ref, wl_ref, bl_ref, wr_ref, o_ref):
    p = jnp.concatenate([p_ref[0], p_ref[1]], axis=1)
    d = d_ref[0] + d_ref[1]
    deg = jnp.maximum(d[:, 0:1], 1.0)
    mean = p / deg
    xf = jnp.concatenate([x_ref[0], x_ref[1]], axis=1)
    for t in range(2):
        acc = lax.dot_general(mean, wl_ref[t], (((1,), (1,)), ((), ())),
                              preferred_element_type=jnp.float32)
        acc = acc + bl_ref[t]
        acc = acc + lax.dot_general(xf, wr_ref[t], (((1,), (1,)), ((), ())),
                                    preferred_element_type=jnp.float32)
        if relu:
            acc = jnp.maximum(acc, 0.0)
        o_ref[t] = acc


def _dense(parts, deg, xs, Wl, bl, Wr, relu):
    R = 2000
    grid = (N_NODES // R,)
    return pl.pallas_call(
        functools.partial(_dense_body, relu),
        grid=grid,
        in_specs=[
            pl.BlockSpec((2, R, HFEATS), lambda i: (0, i, 0)),
            pl.BlockSpec((2, R, L), lambda i: (0, i, 0)),
            pl.BlockSpec((2, R, HFEATS), lambda i: (0, i, 0)),
            pl.BlockSpec((2, HFEATS, FEATS), lambda i: (0, 0, 0)),
            pl.BlockSpec((2, 1, HFEATS), lambda i: (0, 0, 0)),
            pl.BlockSpec((2, HFEATS, FEATS), lambda i: (0, 0, 0)),
        ],
        out_specs=pl.BlockSpec((2, R, HFEATS), lambda i: (0, i, 0)),
        out_shape=jax.ShapeDtypeStruct((2, N_NODES, HFEATS), jnp.float32),
    )(parts, deg, xs, Wl, bl, Wr)


def _split_w(W):
    return W.reshape(2, HFEATS, FEATS)


def _split_b(b):
    return b.reshape(2, 1, HFEATS)


def kernel(x, edge_index, W1l, b1l, W1r, W2l, b2l, W2r):
    xs = jnp.stack([x[:, :HFEATS], x[:, HFEATS:]])
    srcs_a = edge_index[0].reshape(NS, A_NCHUNK, CHUNK)
    dsts_a = edge_index[1].reshape(NS, A_NCHUNK, CHUNK)
    p1, deg = _make_agg_deg()(xs[0], xs[1], srcs_a, dsts_a)
    hs = _dense(p1, deg, xs, _split_w(W1l), _split_b(b1l), _split_w(W1r),
                relu=True)
    p2 = _make_agg()(hs[0], hs[1], srcs_a, dsts_a)
    os = _dense(p2, deg, hs, _split_w(W2l), _split_b(b2l), _split_w(W2r),
                relu=False)
    return jnp.concatenate([os[0], os[1]], axis=1)
